# baseline (speedup 1.0000x reference)
import functools

import jax
import jax.numpy as jnp
from jax import lax
from jax.experimental import pallas as pl
from jax.experimental.pallas import tpu as pltpu
from jax.experimental.pallas import tpu_sc as plsc

B = 4096
S = 200
D = 64
NUM_LABELS = 100

NC = 2
NS = 16
NW = NC * NS
CH_A = 104
CH_B = S - CH_A
ROWS_PER_W = B // NW
LANES = 16
DV = D // LANES

_mesh = plsc.VectorSubcoreMesh(core_axis_name="c", subcore_axis_name="s")


@functools.partial(
    pl.kernel,
    out_type=jax.ShapeDtypeStruct((B, D), jnp.float32),
    mesh=_mesh,
    scratch_types=[
        pltpu.VMEM((ROWS_PER_W, CH_A), jnp.int32),
        pltpu.VMEM((ROWS_PER_W, CH_B), jnp.int32),
        pltpu.VMEM((CH_A, D), jnp.float32),
        pltpu.VMEM((CH_B, D), jnp.float32),
        pltpu.VMEM((ROWS_PER_W, D), jnp.float32),
        pltpu.SemaphoreType.DMA,
        pltpu.SemaphoreType.DMA,
    ],
    compiler_params=pltpu.CompilerParams(use_tc_tiling_on_sc=False),
)
def _pool_sc(tok_hbm, emb_hbm, out_hbm, idx_a, idx_b, buf0, buf1, acc_v,
             sem0, sem1):
    wid = lax.axis_index("s") * NC + lax.axis_index("c")
    bufs = (buf0, buf1)
    sems = (sem0, sem1)

    base_r = wid * ROWS_PER_W
    pltpu.sync_copy(tok_hbm.at[pl.ds(base_r, ROWS_PER_W), pl.ds(0, CH_A)],
                    idx_a)
    pltpu.sync_copy(tok_hbm.at[pl.ds(base_r, ROWS_PER_W), pl.ds(CH_A, CH_B)],
                    idx_b)
    halves = (idx_a, idx_b)

    for p in range(2):
        pltpu.async_copy(emb_hbm.at[halves[p].at[0]], bufs[p], sems[p])

    def pair_body(k, _):
        acc = [jnp.zeros((LANES,), jnp.float32) for _ in range(DV)]
        for p, width in ((0, CH_A), (1, CH_B)):
            pltpu.make_async_copy(
                emb_hbm.at[halves[p].at[k]], bufs[p], sems[p]).wait()
            for t in range(width):
                for j in range(DV):
                    acc[j] = acc[j] + bufs[p][t, pl.ds(j * LANES, LANES)]
            @pl.when(k + 1 < ROWS_PER_W)
            def _():
                pltpu.async_copy(
                    emb_hbm.at[halves[p].at[k + 1]], bufs[p], sems[p])
        for j in range(DV):
            acc_v[k, pl.ds(j * LANES, LANES)] = acc[j]
        return 0

    lax.fori_loop(0, ROWS_PER_W, pair_body, 0)

    pltpu.sync_copy(acc_v, out_hbm.at[pl.ds(wid * ROWS_PER_W, ROWS_PER_W)])


def _cls_tc(pooled_ref, w_ref, b_ref, logits_ref, preds_ref):
    pooled = pooled_ref[...] * (1.0 / S)
    logits = (
        jnp.dot(pooled, w_ref[...], preferred_element_type=jnp.float32)
        + b_ref[...]
    )
    logits_ref[...] = logits
    mx = jnp.max(logits, axis=1, keepdims=True)
    lbl = lax.broadcasted_iota(jnp.int32, logits.shape, 1)
    cand = jnp.where(logits == mx, lbl, NUM_LABELS)
    preds_ref[...] = jnp.min(cand, axis=1, keepdims=True)


_cls_call = pl.pallas_call(
    _cls_tc,
    out_shape=(
        jax.ShapeDtypeStruct((B, NUM_LABELS), jnp.float32),
        jax.ShapeDtypeStruct((B, 1), jnp.int32),
    ),
)


@jax.jit
def kernel(token_ids, emb_table, cls_w, cls_b):
    pooled_sum = _pool_sc(token_ids.astype(jnp.int32), emb_table)
    logits, preds = _cls_call(pooled_sum, cls_w, cls_b.reshape(1, NUM_LABELS))
    return logits, preds.reshape(B)
